# Initial kernel scaffold; baseline (speedup 1.0000x reference)
import jax
import jax.numpy as jnp
from jax.experimental import pallas as pl


def kernel(x, edge_index, eps):
    raise NotImplementedError("write your pallas kernel here")


# speedup vs baseline: 3.2656x; 3.2656x over previous
import functools

import jax
import jax.numpy as jnp
from jax import lax
from jax.experimental import pallas as pl
from jax.experimental.pallas import tpu as pltpu
from jax.experimental.pallas import tpu_sc as plsc

_N = 10000
_D = 128
_E = 320000

_NC = 2
_NS = 16
_NW = _NC * _NS
_CH = 128
_K = 80
_EPAD = _NW * _K * _CH
_RPT = 640
_AROWS = _NS * _RPT
_PAD_ROW = _N


def _sc_partial(x, rows, cols, zeros):
    mesh = plsc.VectorSubcoreMesh(core_axis_name="c", subcore_axis_name="s")

    @functools.partial(
        pl.kernel,
        out_type=jax.ShapeDtypeStruct((_NC, _AROWS, _D), jnp.float32),
        mesh=mesh,
        scratch_types=[
            pltpu.VMEM((_K, _CH), jnp.int32),
            pltpu.VMEM((_K, _CH), jnp.int32),
            pltpu.VMEM((_CH, _D), jnp.float32),
            pltpu.VMEM_SHARED((_AROWS, _D), jnp.float32),
            pltpu.SemaphoreType.DMA,
        ],
    )
    def k(x_hbm, row_hbm, col_hbm, z_hbm, part_hbm, row_v, col_v, buf, acc, sem):
        cid = lax.axis_index("c")
        sid = lax.axis_index("s")
        wid = cid * _NS + sid
        pltpu.sync_copy(z_hbm, acc.at[pl.ds(sid * _RPT, _RPT)])
        pltpu.sync_copy(row_hbm.at[wid], row_v)
        pltpu.sync_copy(col_hbm.at[wid], col_v)
        plsc.subcore_barrier()

        def chunk(j, carry):
            pltpu.async_copy(x_hbm.at[col_v.at[j]], buf, sem).wait()
            pltpu.sync_copy(buf, acc.at[row_v.at[j]], add=True)
            return carry

        lax.fori_loop(0, _K, chunk, 0)

        plsc.subcore_barrier()
        pltpu.sync_copy(acc.at[pl.ds(sid * _RPT, _RPT)],
                        part_hbm.at[cid].at[pl.ds(sid * _RPT, _RPT)])

    return k(x, rows, cols, zeros)


def _combine_body(eps_ref, x_ref, p0_ref, p1_ref, o_ref):
    o_ref[...] = ((1.0 + eps_ref[0, 0]) * x_ref[...]
                  + p0_ref[0] + p1_ref[0])


def _combine(x, part, eps):
    blk = 1000
    return pl.pallas_call(
        _combine_body,
        grid=(_N // blk,),
        in_specs=[
            pl.BlockSpec((1, 1), lambda i: (0, 0)),
            pl.BlockSpec((blk, _D), lambda i: (i, 0)),
            pl.BlockSpec((1, blk, _D), lambda i: (0, i, 0)),
            pl.BlockSpec((1, blk, _D), lambda i: (1, i, 0)),
        ],
        out_specs=pl.BlockSpec((blk, _D), lambda i: (i, 0)),
        out_shape=jax.ShapeDtypeStruct((_N, _D), jnp.float32),
    )(eps.reshape(1, 1), x, part, part)


def kernel(x, edge_index, eps):
    ei = edge_index.astype(jnp.int32)
    pad = _EPAD - _E
    rows = jnp.concatenate(
        [ei[0], jnp.full((pad,), _PAD_ROW, jnp.int32)]).reshape(_NW, _K, _CH)
    cols = jnp.concatenate(
        [ei[1], jnp.zeros((pad,), jnp.int32)]).reshape(_NW, _K, _CH)
    zeros = jnp.zeros((_RPT, _D), jnp.float32)
    part = _sc_partial(x, rows, cols, zeros)
    return _combine(x, part, eps)


# speedup vs baseline: 3.6065x; 1.1044x over previous
import functools

import jax
import jax.numpy as jnp
from jax import lax
from jax.experimental import pallas as pl
from jax.experimental.pallas import tpu as pltpu
from jax.experimental.pallas import tpu_sc as plsc

_N = 10000
_D = 128
_E = 320000

_NC = 2
_NS = 16
_NW = _NC * _NS
_CH = 64
_K = 160
_EPAD = _NW * _K * _CH
_RPT = 640
_AROWS = _NS * _RPT
_PAD_ROW = _N


def _sc_partial(x, rows, cols, zeros):
    mesh = plsc.VectorSubcoreMesh(core_axis_name="c", subcore_axis_name="s")

    @functools.partial(
        pl.kernel,
        out_type=jax.ShapeDtypeStruct((_NC, _AROWS, _D), jnp.float32),
        mesh=mesh,
        scratch_types=[
            [pltpu.VMEM((1, _CH), jnp.int32)] * 4,
            [pltpu.VMEM((1, _CH), jnp.int32)] * 4,
            [pltpu.VMEM((_CH, _D), jnp.float32)] * 4,
            pltpu.VMEM_SHARED((_AROWS, _D), jnp.float32),
            [pltpu.SemaphoreType.DMA] * 4,
            [pltpu.SemaphoreType.DMA] * 4,
            [pltpu.SemaphoreType.DMA] * 4,
            [pltpu.SemaphoreType.DMA] * 4,
        ],
    )
    def k(x_hbm, row_hbm, col_hbm, z_hbm, part_hbm,
          cbufs, rbufs, bufs, acc, icsems, irsems, gsems, ssems):
        cid = lax.axis_index("c")
        sid = lax.axis_index("s")
        wid = cid * _NS + sid
        pltpu.sync_copy(z_hbm, acc.at[pl.ds(sid * _RPT, _RPT)])
        plsc.subcore_barrier()

        def ic_start(j, b):
            pltpu.async_copy(col_hbm.at[wid].at[pl.ds(j, 1)], cbufs[b],
                             icsems[b])

        def ic_wait(j, b):
            pltpu.make_async_copy(col_hbm.at[wid].at[pl.ds(j, 1)], cbufs[b],
                                  icsems[b]).wait()

        def ir_start(j, b):
            pltpu.async_copy(row_hbm.at[wid].at[pl.ds(j, 1)], rbufs[b],
                             irsems[b])

        def ir_wait(j, b):
            pltpu.make_async_copy(row_hbm.at[wid].at[pl.ds(j, 1)], rbufs[b],
                                  irsems[b]).wait()

        def g_start(b):
            pltpu.async_copy(x_hbm.at[cbufs[b].at[0]], bufs[b], gsems[b])

        def g_wait(b):
            pltpu.make_async_copy(x_hbm.at[cbufs[b].at[0]], bufs[b],
                                  gsems[b]).wait()

        def s_start(b):
            pltpu.async_copy(bufs[b], acc.at[rbufs[b].at[0]], ssems[b],
                             add=True)

        def s_wait(b):
            pltpu.make_async_copy(bufs[b], acc.at[rbufs[b].at[0]],
                                  ssems[b]).wait()

        ic_start(0, 0)
        ic_start(1, 1)
        ir_start(0, 0)

        def iter_body(j, b, first, last):
            b1, b2, b3 = (b + 1) % 4, (b + 2) % 4, (b + 3) % 4
            ic_wait(j, b)
            g_start(b)
            if not first or j >= 1:
                ir_wait(j - 1, b3)
                g_wait(b3)
                s_start(b3)
            if not first or j >= 3:
                s_wait(b1)
            if not last or j + 2 <= _K - 1:
                ic_start(j + 2, b2)
            if not last or j + 1 <= _K - 1:
                ir_start(j + 1, b1)

        for j in range(4):
            iter_body(j, j, True, False)

        def group(jj, carry):
            for b in range(4):
                iter_body(jj * 4 + b, b, False, False)
            return carry

        lax.fori_loop(1, _K // 4 - 1, group, 0)

        for b in range(4):
            iter_body(_K - 4 + b, b, False, True)
        ir_wait(_K - 1, (_K - 1) % 4)
        g_wait((_K - 1) % 4)
        s_start((_K - 1) % 4)
        for j in (_K - 3, _K - 2, _K - 1):
            s_wait(j % 4)

        plsc.subcore_barrier()
        pltpu.sync_copy(acc.at[pl.ds(sid * _RPT, _RPT)],
                        part_hbm.at[cid].at[pl.ds(sid * _RPT, _RPT)])

    return k(x, rows, cols, zeros)


def _combine_body(eps_ref, x_ref, p0_ref, p1_ref, o_ref):
    o_ref[...] = ((1.0 + eps_ref[0, 0]) * x_ref[...]
                  + p0_ref[0] + p1_ref[0])


def _combine(x, part, eps):
    blk = 1000
    return pl.pallas_call(
        _combine_body,
        grid=(_N // blk,),
        in_specs=[
            pl.BlockSpec((1, 1), lambda i: (0, 0)),
            pl.BlockSpec((blk, _D), lambda i: (i, 0)),
            pl.BlockSpec((1, blk, _D), lambda i: (0, i, 0)),
            pl.BlockSpec((1, blk, _D), lambda i: (1, i, 0)),
        ],
        out_specs=pl.BlockSpec((blk, _D), lambda i: (i, 0)),
        out_shape=jax.ShapeDtypeStruct((_N, _D), jnp.float32),
    )(eps.reshape(1, 1), x, part, part)


def kernel(x, edge_index, eps):
    ei = edge_index.astype(jnp.int32)
    pad = _EPAD - _E
    rows = jnp.concatenate(
        [ei[0], jnp.full((pad,), _PAD_ROW, jnp.int32)]).reshape(_NW, _K, _CH)
    cols = jnp.concatenate(
        [ei[1], jnp.zeros((pad,), jnp.int32)]).reshape(_NW, _K, _CH)
    zeros = jnp.zeros((_RPT, _D), jnp.float32)
    part = _sc_partial(x, rows, cols, zeros)
    return _combine(x, part, eps)


# speedup vs baseline: 3.8549x; 1.0689x over previous
import functools

import jax
import jax.numpy as jnp
from jax import lax
from jax.experimental import pallas as pl
from jax.experimental.pallas import tpu as pltpu
from jax.experimental.pallas import tpu_sc as plsc

_N = 10000
_D = 128
_E = 320000

_NC = 2
_NS = 16
_NW = _NC * _NS
_CH = 64
_K0 = 240
_K1 = 80
_NCHUNK = _NS * (_K0 + _K1)
_EPAD = _NCHUNK * _CH
_RPT = 640
_AROWS = _NS * _RPT
_PAD_ROW = _N


def _sc_partial(x, rows, cols, zeros):
    mesh = plsc.VectorSubcoreMesh(core_axis_name="c", subcore_axis_name="s")

    @functools.partial(
        pl.kernel,
        out_type=jax.ShapeDtypeStruct((_NC, _AROWS, _D), jnp.float32),
        mesh=mesh,
        scratch_types=[
            [pltpu.VMEM((1, _CH), jnp.int32)] * 4,
            [pltpu.VMEM((1, _CH), jnp.int32)] * 4,
            [pltpu.VMEM((_CH, _D), jnp.float32)] * 4,
            pltpu.VMEM_SHARED((_AROWS, _D), jnp.float32),
            [pltpu.SemaphoreType.DMA] * 4,
            [pltpu.SemaphoreType.DMA] * 4,
            [pltpu.SemaphoreType.DMA] * 4,
            [pltpu.SemaphoreType.DMA] * 4,
        ],
    )
    def k(x_hbm, row_hbm, col_hbm, z_hbm, part_hbm,
          cbufs, rbufs, bufs, acc, icsems, irsems, gsems, ssems):
        cid = lax.axis_index("c")
        sid = lax.axis_index("s")
        pltpu.sync_copy(z_hbm, acc.at[pl.ds(sid * _RPT, _RPT)])
        plsc.subcore_barrier()

        def pipeline(base, kk):
            def ic_start(j, b):
                pltpu.async_copy(col_hbm.at[pl.ds(base + j, 1)], cbufs[b],
                                 icsems[b])

            def ic_wait(j, b):
                pltpu.make_async_copy(col_hbm.at[pl.ds(base + j, 1)],
                                      cbufs[b], icsems[b]).wait()

            def ir_start(j, b):
                pltpu.async_copy(row_hbm.at[pl.ds(base + j, 1)], rbufs[b],
                                 irsems[b])

            def ir_wait(j, b):
                pltpu.make_async_copy(row_hbm.at[pl.ds(base + j, 1)],
                                      rbufs[b], irsems[b]).wait()

            def g_start(b):
                pltpu.async_copy(x_hbm.at[cbufs[b].at[0]], bufs[b], gsems[b])

            def g_wait(b):
                pltpu.make_async_copy(x_hbm.at[cbufs[b].at[0]], bufs[b],
                                      gsems[b]).wait()

            def s_start(b):
                pltpu.async_copy(bufs[b], acc.at[rbufs[b].at[0]], ssems[b],
                                 add=True)

            def s_wait(b):
                pltpu.make_async_copy(bufs[b], acc.at[rbufs[b].at[0]],
                                      ssems[b]).wait()

            ic_start(0, 0)
            ic_start(1, 1)
            ir_start(0, 0)

            def iter_body(j, b, first, last):
                b1, b2, b3 = (b + 1) % 4, (b + 2) % 4, (b + 3) % 4
                ic_wait(j, b)
                g_start(b)
                if not first or b >= 1:
                    ir_wait(j - 1, b3)
                    g_wait(b3)
                    s_start(b3)
                if not first or b >= 3:
                    s_wait(b1)
                if not last or b <= 1:
                    ic_start(j + 2, b2)
                if not last or b <= 2:
                    ir_start(j + 1, b1)

            for b in range(4):
                iter_body(b, b, True, False)

            def group(jj, carry):
                for b in range(4):
                    iter_body(jj * 4 + b, b, False, False)
                return carry

            lax.fori_loop(1, kk // 4 - 1, group, 0)

            for b in range(4):
                iter_body(kk - 4 + b, b, False, True)
            ir_wait(kk - 1, 3)
            g_wait(3)
            s_start(3)
            for b in (1, 2, 3):
                s_wait(b)

        @pl.when(cid == 0)
        def _():
            pipeline(sid * _K0, _K0)

        @pl.when(cid == 1)
        def _():
            pipeline(_NS * _K0 + sid * _K1, _K1)

        plsc.subcore_barrier()
        pltpu.sync_copy(acc.at[pl.ds(sid * _RPT, _RPT)],
                        part_hbm.at[cid].at[pl.ds(sid * _RPT, _RPT)])

    return k(x, rows, cols, zeros)


def _combine_body(eps_ref, x_ref, p0_ref, p1_ref, o_ref):
    o_ref[...] = ((1.0 + eps_ref[0, 0]) * x_ref[...]
                  + p0_ref[0] + p1_ref[0])


def _combine(x, part, eps):
    blk = 1000
    return pl.pallas_call(
        _combine_body,
        grid=(_N // blk,),
        in_specs=[
            pl.BlockSpec((1, 1), lambda i: (0, 0)),
            pl.BlockSpec((blk, _D), lambda i: (i, 0)),
            pl.BlockSpec((1, blk, _D), lambda i: (0, i, 0)),
            pl.BlockSpec((1, blk, _D), lambda i: (1, i, 0)),
        ],
        out_specs=pl.BlockSpec((blk, _D), lambda i: (i, 0)),
        out_shape=jax.ShapeDtypeStruct((_N, _D), jnp.float32),
    )(eps.reshape(1, 1), x, part, part)


def kernel(x, edge_index, eps):
    ei = edge_index.astype(jnp.int32)
    pad = _EPAD - _E
    rows = jnp.concatenate(
        [ei[0], jnp.full((pad,), _PAD_ROW, jnp.int32)]).reshape(_NCHUNK, _CH)
    cols = jnp.concatenate(
        [ei[1], jnp.zeros((pad,), jnp.int32)]).reshape(_NCHUNK, _CH)
    zeros = jnp.zeros((_RPT, _D), jnp.float32)
    part = _sc_partial(x, rows, cols, zeros)
    return _combine(x, part, eps)


# speedup vs baseline: 3.8560x; 1.0003x over previous
import functools

import jax
import jax.numpy as jnp
from jax import lax
from jax.experimental import pallas as pl
from jax.experimental.pallas import tpu as pltpu
from jax.experimental.pallas import tpu_sc as plsc

_N = 10000
_D = 128
_E = 320000

_NC = 2
_NS = 16
_NW = _NC * _NS
_CH = 64
_K0 = 240
_K1 = 80
_NCHUNK = _NS * (_K0 + _K1)
_EPAD = _NCHUNK * _CH
_RPT = 640
_AROWS = _NS * _RPT
_PAD_ROW = _N


def _sc_partial(x, rows, cols, zeros):
    mesh = plsc.VectorSubcoreMesh(core_axis_name="c", subcore_axis_name="s")

    @functools.partial(
        pl.kernel,
        out_type=jax.ShapeDtypeStruct((_NC, _AROWS, _D), jnp.float32),
        mesh=mesh,
        scratch_types=[
            [pltpu.VMEM((1, _CH), jnp.int32)] * 4,
            [pltpu.VMEM((1, _CH), jnp.int32)] * 4,
            [pltpu.VMEM((_CH, _D), jnp.float32)] * 4,
            pltpu.VMEM_SHARED((_AROWS, _D), jnp.float32),
            [pltpu.SemaphoreType.DMA] * 4,
            [pltpu.SemaphoreType.DMA] * 4,
            [pltpu.SemaphoreType.DMA] * 4,
            [pltpu.SemaphoreType.DMA] * 4,
        ],
    )
    def k(x_hbm, row_hbm, col_hbm, z_hbm, part_hbm,
          cbufs, rbufs, bufs, acc, icsems, irsems, gsems, ssems):
        cid = lax.axis_index("c")
        sid = lax.axis_index("s")
        with jax.named_scope("zero_init"):
            pltpu.sync_copy(z_hbm, acc.at[pl.ds(sid * _RPT, _RPT)])
            plsc.subcore_barrier()

        def pipeline(base, kk):
            def ic_start(j, b):
                pltpu.async_copy(col_hbm.at[pl.ds(base + j, 1)], cbufs[b],
                                 icsems[b])

            def ic_wait(j, b):
                pltpu.make_async_copy(col_hbm.at[pl.ds(base + j, 1)],
                                      cbufs[b], icsems[b]).wait()

            def ir_start(j, b):
                pltpu.async_copy(row_hbm.at[pl.ds(base + j, 1)], rbufs[b],
                                 irsems[b])

            def ir_wait(j, b):
                pltpu.make_async_copy(row_hbm.at[pl.ds(base + j, 1)],
                                      rbufs[b], irsems[b]).wait()

            def g_start(b):
                pltpu.async_copy(x_hbm.at[cbufs[b].at[0]], bufs[b], gsems[b])

            def g_wait(b):
                pltpu.make_async_copy(x_hbm.at[cbufs[b].at[0]], bufs[b],
                                      gsems[b]).wait()

            def s_start(b):
                pltpu.async_copy(bufs[b], acc.at[rbufs[b].at[0]], ssems[b],
                                 add=True)

            def s_wait(b):
                pltpu.make_async_copy(bufs[b], acc.at[rbufs[b].at[0]],
                                      ssems[b]).wait()

            ic_start(0, 0)
            ic_start(1, 1)
            ir_start(0, 0)

            def iter_body(j, b, first, last):
                b1, b2, b3 = (b + 1) % 4, (b + 2) % 4, (b + 3) % 4
                ic_wait(j, b)
                g_start(b)
                if not first or b >= 1:
                    ir_wait(j - 1, b3)
                    g_wait(b3)
                    s_start(b3)
                if not first or b >= 3:
                    s_wait(b1)
                if not last or b <= 1:
                    ic_start(j + 2, b2)
                if not last or b <= 2:
                    ir_start(j + 1, b1)

            for b in range(4):
                iter_body(b, b, True, False)

            def group(jj, carry):
                for b in range(4):
                    iter_body(jj * 4 + b, b, False, False)
                return carry

            lax.fori_loop(1, kk // 4 - 1, group, 0)

            for b in range(4):
                iter_body(kk - 4 + b, b, False, True)
            ir_wait(kk - 1, 3)
            g_wait(3)
            s_start(3)
            for b in (1, 2, 3):
                s_wait(b)

        with jax.named_scope("edges"):
            @pl.when(cid == 0)
            def _():
                pipeline(sid * _K0, _K0)

            @pl.when(cid == 1)
            def _():
                pipeline(_NS * _K0 + sid * _K1, _K1)

        with jax.named_scope("writeback"):
            plsc.subcore_barrier()
            pltpu.sync_copy(acc.at[pl.ds(sid * _RPT, _RPT)],
                            part_hbm.at[cid].at[pl.ds(sid * _RPT, _RPT)])

    return k(x, rows, cols, zeros)


def _combine_body(eps_ref, x_ref, p0_ref, p1_ref, o_ref):
    o_ref[...] = ((1.0 + eps_ref[0, 0]) * x_ref[...]
                  + p0_ref[0] + p1_ref[0])


def _combine(x, part, eps):
    blk = 1000
    return pl.pallas_call(
        _combine_body,
        grid=(_N // blk,),
        in_specs=[
            pl.BlockSpec((1, 1), lambda i: (0, 0)),
            pl.BlockSpec((blk, _D), lambda i: (i, 0)),
            pl.BlockSpec((1, blk, _D), lambda i: (0, i, 0)),
            pl.BlockSpec((1, blk, _D), lambda i: (1, i, 0)),
        ],
        out_specs=pl.BlockSpec((blk, _D), lambda i: (i, 0)),
        out_shape=jax.ShapeDtypeStruct((_N, _D), jnp.float32),
    )(eps.reshape(1, 1), x, part, part)


def kernel(x, edge_index, eps):
    ei = edge_index.astype(jnp.int32)
    pad = _EPAD - _E
    rows = jnp.concatenate(
        [ei[0], jnp.full((pad,), _PAD_ROW, jnp.int32)]).reshape(_NCHUNK, _CH)
    cols = jnp.concatenate(
        [ei[1], jnp.zeros((pad,), jnp.int32)]).reshape(_NCHUNK, _CH)
    zeros = jnp.zeros((_RPT, _D), jnp.float32)
    part = _sc_partial(x, rows, cols, zeros)
    return _combine(x, part, eps)


# speedup vs baseline: 12.4648x; 3.2326x over previous
import functools

import jax
import jax.numpy as jnp
from jax import lax
from jax.experimental import pallas as pl
from jax.experimental.pallas import tpu as pltpu
from jax.experimental.pallas import tpu_sc as plsc

_N = 10000
_D = 128
_E = 320000

_NC = 2
_NS = 16
_NW = _NC * _NS
_CH = 64
_K = 160
_NCHUNK = _NW * _K
_EPAD = _NCHUNK * _CH
_RPT = 640
_AROWS = _NS * _RPT
_PAD_ROW = _N


def _sc_partial(x, rows, cols, zeros):
    mesh = plsc.VectorSubcoreMesh(core_axis_name="c", subcore_axis_name="s")

    @functools.partial(
        pl.kernel,
        out_type=jax.ShapeDtypeStruct((_NC, _AROWS, _D), jnp.float32),
        mesh=mesh,
        scratch_types=[
            [pltpu.VMEM((1, _CH), jnp.int32)] * 4,
            [pltpu.VMEM((1, _CH), jnp.int32)] * 4,
            [pltpu.VMEM((_CH, _D), jnp.float32)] * 4,
            pltpu.VMEM_SHARED((_AROWS, _D), jnp.float32),
            [pltpu.SemaphoreType.DMA] * 4,
            [pltpu.SemaphoreType.DMA] * 4,
            [pltpu.SemaphoreType.DMA] * 4,
            [pltpu.SemaphoreType.DMA] * 4,
        ],
    )
    def k(x_hbm, row_hbm, col_hbm, z_hbm, part_hbm,
          cbufs, rbufs, bufs, acc, icsems, irsems, gsems, ssems):
        cid = lax.axis_index("c")
        sid = lax.axis_index("s")
        with jax.named_scope("zero_init"):
            pltpu.sync_copy(z_hbm, acc.at[pl.ds(sid * _RPT, _RPT)])
            plsc.subcore_barrier()

        def pipeline(base, kk):
            def ic_start(j, b):
                pltpu.async_copy(col_hbm.at[pl.ds(base + j, 1)], cbufs[b],
                                 icsems[b])

            def ic_wait(j, b):
                pltpu.make_async_copy(col_hbm.at[pl.ds(base + j, 1)],
                                      cbufs[b], icsems[b]).wait()

            def ir_start(j, b):
                pltpu.async_copy(row_hbm.at[pl.ds(base + j, 1)], rbufs[b],
                                 irsems[b])

            def ir_wait(j, b):
                pltpu.make_async_copy(row_hbm.at[pl.ds(base + j, 1)],
                                      rbufs[b], irsems[b]).wait()

            def g_start(b):
                pltpu.async_copy(x_hbm.at[cbufs[b].at[0]], bufs[b], gsems[b])

            def g_wait(b):
                pltpu.make_async_copy(x_hbm.at[cbufs[b].at[0]], bufs[b],
                                      gsems[b]).wait()

            def s_start(b):
                pltpu.async_copy(bufs[b], acc.at[rbufs[b].at[0]], ssems[b],
                                 add=True)

            def s_wait(b):
                pltpu.make_async_copy(bufs[b], acc.at[rbufs[b].at[0]],
                                      ssems[b]).wait()

            ic_start(0, 0)
            ic_start(1, 1)
            ir_start(0, 0)

            def iter_body(j, b, first, last):
                b1, b2, b3 = (b + 1) % 4, (b + 2) % 4, (b + 3) % 4
                ic_wait(j, b)
                g_start(b)
                if not first or b >= 1:
                    ir_wait(j - 1, b3)
                    g_wait(b3)
                    s_start(b3)
                if not first or b >= 3:
                    s_wait(b1)
                if not last or b <= 1:
                    ic_start(j + 2, b2)
                if not last or b <= 2:
                    ir_start(j + 1, b1)

            for b in range(4):
                iter_body(b, b, True, False)

            def group(jj, carry):
                for b in range(4):
                    iter_body(jj * 4 + b, b, False, False)
                return carry

            lax.fori_loop(1, kk // 4 - 1, group, 0)

            for b in range(4):
                iter_body(kk - 4 + b, b, False, True)
            ir_wait(kk - 1, 3)
            g_wait(3)
            s_start(3)
            for b in (1, 2, 3):
                s_wait(b)

        with jax.named_scope("edges"):
            pipeline((cid * _NS + sid) * _K, _K)

        with jax.named_scope("writeback"):
            plsc.subcore_barrier()
            pltpu.sync_copy(acc.at[pl.ds(sid * _RPT, _RPT)],
                            part_hbm.at[cid].at[pl.ds(sid * _RPT, _RPT)])

    return k(x, rows, cols, zeros)


def _combine_body(eps_ref, x_ref, p0_ref, p1_ref, o_ref):
    o_ref[...] = ((1.0 + eps_ref[0, 0]) * x_ref[...]
                  + p0_ref[0] + p1_ref[0])


def _combine(x, part, eps):
    blk = 1000
    return pl.pallas_call(
        _combine_body,
        grid=(_N // blk,),
        in_specs=[
            pl.BlockSpec((1, 1), lambda i: (0, 0)),
            pl.BlockSpec((blk, _D), lambda i: (i, 0)),
            pl.BlockSpec((1, blk, _D), lambda i: (0, i, 0)),
            pl.BlockSpec((1, blk, _D), lambda i: (1, i, 0)),
        ],
        out_specs=pl.BlockSpec((blk, _D), lambda i: (i, 0)),
        out_shape=jax.ShapeDtypeStruct((_N, _D), jnp.float32),
    )(eps.reshape(1, 1), x, part, part)


def kernel(x, edge_index, eps):
    ei = edge_index.astype(jnp.int32)
    pad = _EPAD - _E
    pad_rows = _PAD_ROW + (jnp.arange(pad, dtype=jnp.int32) % (_AROWS - _N))
    pad_cols = jnp.arange(pad, dtype=jnp.int32) % _N
    rows = jnp.concatenate([ei[0], pad_rows]).reshape(_NCHUNK, _CH)
    cols = jnp.concatenate([ei[1], pad_cols]).reshape(_NCHUNK, _CH)
    zeros = jnp.zeros((_RPT, _D), jnp.float32)
    part = _sc_partial(x, rows, cols, zeros)
    return _combine(x, part, eps)


# speedup vs baseline: 13.8698x; 1.1127x over previous
import functools

import jax
import jax.numpy as jnp
from jax import lax
from jax.experimental import pallas as pl
from jax.experimental.pallas import tpu as pltpu
from jax.experimental.pallas import tpu_sc as plsc

_N = 10000
_D = 128
_E = 320000

_NC = 2
_NS = 16
_NW = _NC * _NS
_CH = 64
_NCHUNK = _E // _CH
_KBIG = 160
_KSML = 156
_RPT = 640
_AROWS = _NS * _RPT


def _sc_partial(x, edges, zeros):
    mesh = plsc.VectorSubcoreMesh(core_axis_name="c", subcore_axis_name="s")

    @functools.partial(
        pl.kernel,
        out_type=jax.ShapeDtypeStruct((_NC, _AROWS, _D), jnp.float32),
        mesh=mesh,
        scratch_types=[
            [pltpu.VMEM((_CH,), jnp.int32)] * 4,
            [pltpu.VMEM((_CH,), jnp.int32)] * 4,
            [pltpu.VMEM((_CH, _D), jnp.float32)] * 4,
            pltpu.VMEM_SHARED((_AROWS, _D), jnp.float32),
            [pltpu.SemaphoreType.DMA] * 4,
            [pltpu.SemaphoreType.DMA] * 4,
            [pltpu.SemaphoreType.DMA] * 4,
            [pltpu.SemaphoreType.DMA] * 4,
        ],
    )
    def k(x_hbm, e_hbm, z_hbm, part_hbm,
          cbufs, rbufs, bufs, acc, icsems, irsems, gsems, ssems):
        cid = lax.axis_index("c")
        sid = lax.axis_index("s")
        wid = cid * _NS + sid
        pltpu.sync_copy(z_hbm, acc.at[pl.ds(sid * _RPT, _RPT)])
        plsc.subcore_barrier()

        def pipeline(base, kk):
            def ic_start(j, b):
                pltpu.async_copy(
                    e_hbm.at[1].at[pl.ds((base + j) * _CH, _CH)],
                    cbufs[b], icsems[b])

            def ic_wait(j, b):
                pltpu.make_async_copy(
                    e_hbm.at[1].at[pl.ds((base + j) * _CH, _CH)],
                    cbufs[b], icsems[b]).wait()

            def ir_start(j, b):
                pltpu.async_copy(
                    e_hbm.at[0].at[pl.ds((base + j) * _CH, _CH)],
                    rbufs[b], irsems[b])

            def ir_wait(j, b):
                pltpu.make_async_copy(
                    e_hbm.at[0].at[pl.ds((base + j) * _CH, _CH)],
                    rbufs[b], irsems[b]).wait()

            def g_start(b):
                pltpu.async_copy(x_hbm.at[cbufs[b]], bufs[b], gsems[b])

            def g_wait(b):
                pltpu.make_async_copy(x_hbm.at[cbufs[b]], bufs[b],
                                      gsems[b]).wait()

            def s_start(b):
                pltpu.async_copy(bufs[b], acc.at[rbufs[b]], ssems[b],
                                 add=True)

            def s_wait(b):
                pltpu.make_async_copy(bufs[b], acc.at[rbufs[b]],
                                      ssems[b]).wait()

            ic_start(0, 0)
            ic_start(1, 1)
            ir_start(0, 0)

            def iter_body(j, b, first, last):
                b1, b2, b3 = (b + 1) % 4, (b + 2) % 4, (b + 3) % 4
                ic_wait(j, b)
                g_start(b)
                if not first or b >= 1:
                    ir_wait(j - 1, b3)
                    g_wait(b3)
                    s_start(b3)
                if not first or b >= 3:
                    s_wait(b1)
                if not last or b <= 1:
                    ic_start(j + 2, b2)
                if not last or b <= 2:
                    ir_start(j + 1, b1)

            for b in range(4):
                iter_body(b, b, True, False)

            def group(jj, carry):
                for b in range(4):
                    iter_body(jj * 4 + b, b, False, False)
                return carry

            lax.fori_loop(1, kk // 4 - 1, group, 0)

            for b in range(4):
                iter_body(kk - 4 + b, b, False, True)
            ir_wait(kk - 1, 3)
            g_wait(3)
            s_start(3)
            for b in (1, 2, 3):
                s_wait(b)

        @pl.when(wid < 2)
        def _():
            pipeline(wid * _KBIG, _KBIG)

        @pl.when(wid >= 2)
        def _():
            pipeline(2 * _KBIG + (wid - 2) * _KSML, _KSML)

        plsc.subcore_barrier()
        pltpu.sync_copy(acc.at[pl.ds(sid * _RPT, _RPT)],
                        part_hbm.at[cid].at[pl.ds(sid * _RPT, _RPT)])

    return k(x, edges, zeros)


def _combine_body(eps_ref, x_ref, p0_ref, p1_ref, o_ref):
    o_ref[...] = ((1.0 + eps_ref[0, 0]) * x_ref[...]
                  + p0_ref[0] + p1_ref[0])


def _combine(x, part, eps):
    blk = 1000
    return pl.pallas_call(
        _combine_body,
        grid=(_N // blk,),
        in_specs=[
            pl.BlockSpec((1, 1), lambda i: (0, 0)),
            pl.BlockSpec((blk, _D), lambda i: (i, 0)),
            pl.BlockSpec((1, blk, _D), lambda i: (0, i, 0)),
            pl.BlockSpec((1, blk, _D), lambda i: (1, i, 0)),
        ],
        out_specs=pl.BlockSpec((blk, _D), lambda i: (i, 0)),
        out_shape=jax.ShapeDtypeStruct((_N, _D), jnp.float32),
    )(eps.reshape(1, 1), x, part, part)


def kernel(x, edge_index, eps):
    edges = edge_index.astype(jnp.int32)
    zeros = jnp.zeros((_RPT, _D), jnp.float32)
    part = _sc_partial(x, edges, zeros)
    return _combine(x, part, eps)


# speedup vs baseline: 14.1583x; 1.0208x over previous
import functools

import jax
import jax.numpy as jnp
from jax import lax
from jax.experimental import pallas as pl
from jax.experimental.pallas import tpu as pltpu
from jax.experimental.pallas import tpu_sc as plsc

_N = 10000
_D = 128
_E = 320000

_NC = 2
_NS = 16
_NW = _NC * _NS
_CH = 64
_NCHUNK = _E // _CH
_KBIG = 160
_KSML = 156
_RPT = 640
_AROWS = _NS * _RPT


def _sc_partial(x, edges, zeros):
    mesh = plsc.VectorSubcoreMesh(core_axis_name="c", subcore_axis_name="s")

    @functools.partial(
        pl.kernel,
        out_type=jax.ShapeDtypeStruct((_NC, _AROWS, _D), jnp.float32),
        mesh=mesh,
        scratch_types=[
            [pltpu.VMEM((_CH,), jnp.int32)] * 4,
            [pltpu.VMEM((_CH,), jnp.int32)] * 4,
            [pltpu.VMEM((_CH, _D), jnp.float32)] * 4,
            pltpu.VMEM_SHARED((_AROWS, _D), jnp.float32),
            [pltpu.SemaphoreType.DMA] * 4,
            [pltpu.SemaphoreType.DMA] * 4,
            [pltpu.SemaphoreType.DMA] * 4,
            [pltpu.SemaphoreType.DMA] * 4,
        ],
    )
    def k(x_hbm, e_hbm, z_hbm, part_hbm,
          cbufs, rbufs, bufs, acc, icsems, irsems, gsems, ssems):
        cid = lax.axis_index("c")
        sid = lax.axis_index("s")
        pltpu.sync_copy(z_hbm, acc.at[pl.ds(sid * _RPT, _RPT)])
        plsc.subcore_barrier()

        def pipeline(base, kk):
            def ic_start(j, b):
                pltpu.async_copy(
                    e_hbm.at[1].at[pl.ds((base + j) * _CH, _CH)],
                    cbufs[b], icsems[b])

            def ic_wait(j, b):
                pltpu.make_async_copy(
                    e_hbm.at[1].at[pl.ds((base + j) * _CH, _CH)],
                    cbufs[b], icsems[b]).wait()

            def ir_start(j, b):
                pltpu.async_copy(
                    e_hbm.at[0].at[pl.ds((base + j) * _CH, _CH)],
                    rbufs[b], irsems[b])

            def ir_wait(j, b):
                pltpu.make_async_copy(
                    e_hbm.at[0].at[pl.ds((base + j) * _CH, _CH)],
                    rbufs[b], irsems[b]).wait()

            def g_start(b):
                pltpu.async_copy(x_hbm.at[cbufs[b]], bufs[b], gsems[b])

            def g_wait(b):
                pltpu.make_async_copy(x_hbm.at[cbufs[b]], bufs[b],
                                      gsems[b]).wait()

            def s_start(b):
                pltpu.async_copy(bufs[b], acc.at[rbufs[b]], ssems[b],
                                 add=True)

            def s_wait(b):
                pltpu.make_async_copy(bufs[b], acc.at[rbufs[b]],
                                      ssems[b]).wait()

            ic_start(0, 0)
            ic_start(1, 1)
            ir_start(0, 0)

            def iter_body(j, b, first, last):
                b1, b2, b3 = (b + 1) % 4, (b + 2) % 4, (b + 3) % 4
                ic_wait(j, b)
                g_start(b)
                if not first or b >= 1:
                    ir_wait(j - 1, b3)
                    g_wait(b3)
                    s_start(b3)
                if not first or b >= 3:
                    s_wait(b1)
                if not last or b <= 1:
                    ic_start(j + 2, b2)
                if not last or b <= 2:
                    ir_start(j + 1, b1)

            for b in range(4):
                iter_body(b, b, True, False)

            def group(jj, carry):
                for b in range(4):
                    iter_body(jj * 4 + b, b, False, False)
                return carry

            lax.fori_loop(1, kk // 4 - 1, group, 0)

            for b in range(4):
                iter_body(kk - 4 + b, b, False, True)
            ir_wait(kk - 1, 3)
            g_wait(3)
            s_start(3)
            for b in (1, 2, 3):
                s_wait(b)

        half = cid * (_NCHUNK // _NC)

        @pl.when(sid < 1)
        def _():
            pipeline(half, _KBIG)

        @pl.when(sid >= 1)
        def _():
            pipeline(half + _KBIG + (sid - 1) * _KSML, _KSML)

        plsc.subcore_barrier()
        pltpu.sync_copy(acc.at[pl.ds(sid * _RPT, _RPT)],
                        part_hbm.at[cid].at[pl.ds(sid * _RPT, _RPT)])

    return k(x, edges, zeros)


def _combine_body(eps_ref, x_ref, p0_ref, p1_ref, o_ref):
    o_ref[...] = ((1.0 + eps_ref[0, 0]) * x_ref[...]
                  + p0_ref[0] + p1_ref[0])


def _combine(x, part, eps):
    blk = 2000
    return pl.pallas_call(
        _combine_body,
        grid=(_N // blk,),
        in_specs=[
            pl.BlockSpec((1, 1), lambda i: (0, 0)),
            pl.BlockSpec((blk, _D), lambda i: (i, 0)),
            pl.BlockSpec((1, blk, _D), lambda i: (0, i, 0)),
            pl.BlockSpec((1, blk, _D), lambda i: (1, i, 0)),
        ],
        out_specs=pl.BlockSpec((blk, _D), lambda i: (i, 0)),
        out_shape=jax.ShapeDtypeStruct((_N, _D), jnp.float32),
    )(eps.reshape(1, 1), x, part, part)


def kernel(x, edge_index, eps):
    edges = edge_index.astype(jnp.int32)
    zeros = jnp.zeros((_RPT, _D), jnp.float32)
    part = _sc_partial(x, edges, zeros)
    return _combine(x, part, eps)
